# baseline (speedup 1.0000x reference)
import jax
import jax.numpy as jnp
from jax.experimental import pallas as pl
from jax.experimental.pallas import tpu as pltpu

_BINS = 10
_SCALE = 10.0 - 0.0001
_ROWS = 16384
_COLS = 1024
_BLOCK_ROWS = 512
_N_STEPS = _ROWS // _BLOCK_ROWS
_ROW_STRIPS = _BLOCK_ROWS // 8
_LANE_CHUNKS = _COLS // 128
_LOG2E = 1.4426950408889634
_LN2 = 0.6931471805599453


def _ghm_kernel(x_ref, t_ref, out_ref, cnt_ref, sum_ref):
    step = pl.program_id(0)

    @pl.when(step == 0)
    def _init():
        cnt_ref[...] = jnp.zeros_like(cnt_ref)
        sum_ref[...] = jnp.zeros_like(sum_ref)

    zeros = jnp.zeros((8, 128), jnp.float32)

    def slab_body(r, c, accs):
        cnts, sums = accs
        x = x_ref[pl.ds(8 * r, 8), pl.ds(128 * c, 128)]
        t = t_ref[pl.ds(8 * r, 8), pl.ds(128 * c, 128)]
        ax = jnp.abs(x)
        e = jnp.exp2(ax * (-_LOG2E))
        one_pe = e + 1.0
        d = 1.0 / one_pe
        sig = jnp.where(x >= 0.0, d, 1.0 - d)
        g = jnp.abs(sig - t)
        binf = jnp.floor(g * _SCALE)
        bce = jnp.maximum(x, 0.0) - x * t + jnp.log2(one_pe) * _LN2
        new_cnts = []
        new_sums = []
        for b in range(_BINS):
            mask = binf == jnp.float32(b)
            new_cnts.append(cnts[b] + jnp.where(mask, 1.0, zeros))
            new_sums.append(sums[b] + jnp.where(mask, bce, zeros))
        return new_cnts, new_sums

    def row_body(r, accs):
        return jax.lax.fori_loop(
            0, _LANE_CHUNKS, lambda c, a: slab_body(r, c, a), accs)

    init = ([zeros] * _BINS, [zeros] * _BINS)
    cnts, sums = jax.lax.fori_loop(0, _ROW_STRIPS, row_body, init)

    for b in range(_BINS):
        cnt_ref[pl.ds(8 * b, 8), :] += cnts[b]
        sum_ref[pl.ds(8 * b, 8), :] += sums[b]

    @pl.when(step == _N_STEPS - 1)
    def _finish():
        counts = [jnp.sum(cnt_ref[pl.ds(8 * b, 8), :]) for b in range(_BINS)]
        bsums = [jnp.sum(sum_ref[pl.ds(8 * b, 8), :]) for b in range(_BINS)]
        nonempty = jnp.float32(0.0)
        for b in range(_BINS):
            nonempty += jnp.where(counts[b] > 0.0, jnp.float32(1.0),
                                  jnp.float32(0.0))
        loss = jnp.float32(0.0)
        for b in range(_BINS):
            gd = jnp.maximum(counts[b] * nonempty, jnp.float32(0.0001))
            loss += bsums[b] / gd
        out_ref[...] = jnp.full((8, 128), loss, dtype=jnp.float32)


@jax.jit
def kernel(x, target):
    out = pl.pallas_call(
        _ghm_kernel,
        grid=(_N_STEPS,),
        in_specs=[
            pl.BlockSpec((_BLOCK_ROWS, _COLS), lambda i: (i, 0)),
            pl.BlockSpec((_BLOCK_ROWS, _COLS), lambda i: (i, 0)),
        ],
        out_specs=pl.BlockSpec((8, 128), lambda i: (0, 0)),
        out_shape=jax.ShapeDtypeStruct((8, 128), jnp.float32),
        scratch_shapes=[
            pltpu.VMEM((8 * _BINS, 128), jnp.float32),
            pltpu.VMEM((8 * _BINS, 128), jnp.float32),
        ],
    )(x, target)
    return out[0, 0]


# speedup vs baseline: 2.7971x; 2.7971x over previous
import jax
import jax.numpy as jnp
from jax.experimental import pallas as pl
from jax.experimental.pallas import tpu as pltpu

_BINS = 10
_SCALE = 10.0 - 0.0001
_ROWS = 16384
_COLS = 1024
_BLOCK_ROWS = 64
_N_STEPS = _ROWS // _BLOCK_ROWS
_LOG2E = 1.4426950408889634
_LN2 = 0.6931471805599453


def _ghm_kernel(x_ref, t_ref, out_ref, cnt_ref, sum_ref):
    step = pl.program_id(0)

    @pl.when(step == 0)
    def _init():
        cnt_ref[...] = jnp.zeros_like(cnt_ref)
        sum_ref[...] = jnp.zeros_like(sum_ref)

    zeroh = jnp.zeros((16, 128), jnp.bfloat16)
    oneh = jnp.bfloat16(1.0)
    acc_c = [zeroh] * _BINS
    acc_s = [zeroh] * _BINS

    for i in range(_BLOCK_ROWS // 16):
        for j in range(_COLS // 128):
            x = x_ref[pl.ds(16 * i, 16), pl.ds(128 * j, 128)]
            t = t_ref[pl.ds(16 * i, 16), pl.ds(128 * j, 128)]
            ax = jnp.abs(x)
            e = jnp.exp2(ax * (-_LOG2E))
            one_pe = e + 1.0
            d = 1.0 / one_pe
            sig = jnp.where(x >= 0.0, d, 1.0 - d)
            g = jnp.abs(sig - t)
            binf = jnp.floor(g * _SCALE)
            bce = jnp.maximum(x, 0.0) - x * t + jnp.log2(one_pe) * _LN2
            binh = binf.astype(jnp.bfloat16)
            bceh = bce.astype(jnp.bfloat16)
            for b in range(_BINS):
                mask = binh == jnp.bfloat16(b)
                acc_c[b] = acc_c[b] + jnp.where(mask, oneh, zeroh)
                acc_s[b] = acc_s[b] + jnp.where(mask, bceh, zeroh)

    for b in range(_BINS):
        cnt_ref[pl.ds(16 * b, 16), :] += acc_c[b].astype(jnp.float32)
        sum_ref[pl.ds(16 * b, 16), :] += acc_s[b].astype(jnp.float32)

    @pl.when(step == _N_STEPS - 1)
    def _finish():
        counts = [jnp.sum(cnt_ref[pl.ds(16 * b, 16), :]) for b in range(_BINS)]
        bsums = [jnp.sum(sum_ref[pl.ds(16 * b, 16), :]) for b in range(_BINS)]
        nonempty = jnp.float32(0.0)
        for b in range(_BINS):
            nonempty += jnp.where(counts[b] > 0.0, jnp.float32(1.0),
                                  jnp.float32(0.0))
        loss = jnp.float32(0.0)
        for b in range(_BINS):
            gd = jnp.maximum(counts[b] * nonempty, jnp.float32(0.0001))
            loss += bsums[b] / gd
        out_ref[...] = jnp.full((8, 128), loss, dtype=jnp.float32)


@jax.jit
def kernel(x, target):
    out = pl.pallas_call(
        _ghm_kernel,
        grid=(_N_STEPS,),
        in_specs=[
            pl.BlockSpec((_BLOCK_ROWS, _COLS), lambda i: (i, 0)),
            pl.BlockSpec((_BLOCK_ROWS, _COLS), lambda i: (i, 0)),
        ],
        out_specs=pl.BlockSpec((8, 128), lambda i: (0, 0)),
        out_shape=jax.ShapeDtypeStruct((8, 128), jnp.float32),
        scratch_shapes=[
            pltpu.VMEM((16 * _BINS, 128), jnp.float32),
            pltpu.VMEM((16 * _BINS, 128), jnp.float32),
        ],
    )(x, target)
    return out[0, 0]


# speedup vs baseline: 3.9501x; 1.4122x over previous
import jax
import jax.numpy as jnp
from jax.experimental import pallas as pl
from jax.experimental.pallas import tpu as pltpu

_BINS = 10
_SCALE = 10.0 - 0.0001
_ROWS = 16384
_COLS = 1024
_BLOCK_ROWS = 128
_N_STEPS = _ROWS // _BLOCK_ROWS
_LOG2E = 1.4426950408889634
_LN2 = 0.6931471805599453


def _ghm_kernel(x_ref, t_ref, out_ref, cnt_ref, sum_ref):
    step = pl.program_id(0)

    @pl.when(step == 0)
    def _init():
        cnt_ref[...] = jnp.zeros_like(cnt_ref)
        sum_ref[...] = jnp.zeros_like(sum_ref)

    zeroh = jnp.zeros((16, 128), jnp.bfloat16)
    oneh = jnp.bfloat16(1.0)
    acc_c = [zeroh] * _BINS
    acc_s = [zeroh] * _BINS

    for i in range(_BLOCK_ROWS // 16):
        for j in range(_COLS // 128):
            x = x_ref[pl.ds(16 * i, 16), pl.ds(128 * j, 128)]
            t = t_ref[pl.ds(16 * i, 16), pl.ds(128 * j, 128)]
            ax = jnp.abs(x)
            e = jnp.exp2(ax * (-_LOG2E))
            one_pe = e + 1.0
            d = 1.0 / one_pe
            sig = jnp.where(x >= 0.0, d, 1.0 - d)
            g = jnp.abs(sig - t)
            binf = jnp.floor(g * _SCALE)
            bce = jnp.maximum(x, 0.0) - x * t + jnp.log2(one_pe) * _LN2
            binh = binf.astype(jnp.bfloat16)
            bceh = bce.astype(jnp.bfloat16)
            for b in range(_BINS):
                mask = binh == jnp.bfloat16(b)
                acc_c[b] = acc_c[b] + jnp.where(mask, oneh, zeroh)
                acc_s[b] = acc_s[b] + jnp.where(mask, bceh, zeroh)

    for b in range(_BINS):
        cnt_ref[pl.ds(16 * b, 16), :] += acc_c[b].astype(jnp.float32)
        sum_ref[pl.ds(16 * b, 16), :] += acc_s[b].astype(jnp.float32)

    @pl.when(step == _N_STEPS - 1)
    def _finish():
        counts = [jnp.sum(cnt_ref[pl.ds(16 * b, 16), :]) for b in range(_BINS)]
        bsums = [jnp.sum(sum_ref[pl.ds(16 * b, 16), :]) for b in range(_BINS)]
        nonempty = jnp.float32(0.0)
        for b in range(_BINS):
            nonempty += jnp.where(counts[b] > 0.0, jnp.float32(1.0),
                                  jnp.float32(0.0))
        loss = jnp.float32(0.0)
        for b in range(_BINS):
            gd = jnp.maximum(counts[b] * nonempty, jnp.float32(0.0001))
            loss += bsums[b] / gd
        out_ref[...] = jnp.full((8, 128), loss, dtype=jnp.float32)


@jax.jit
def kernel(x, target):
    out = pl.pallas_call(
        _ghm_kernel,
        grid=(_N_STEPS,),
        in_specs=[
            pl.BlockSpec((_BLOCK_ROWS, _COLS), lambda i: (i, 0)),
            pl.BlockSpec((_BLOCK_ROWS, _COLS), lambda i: (i, 0)),
        ],
        out_specs=pl.BlockSpec((8, 128), lambda i: (0, 0)),
        out_shape=jax.ShapeDtypeStruct((8, 128), jnp.float32),
        scratch_shapes=[
            pltpu.VMEM((16 * _BINS, 128), jnp.float32),
            pltpu.VMEM((16 * _BINS, 128), jnp.float32),
        ],
    )(x, target)
    return out[0, 0]


# speedup vs baseline: 4.5409x; 1.1496x over previous
import jax
import jax.numpy as jnp
from jax.experimental import pallas as pl
from jax.experimental.pallas import tpu as pltpu

_BINS = 10
_SCALE = 10.0 - 0.0001
_ROWS = 16384
_COLS = 1024
_BLOCK_ROWS = 256
_N_STEPS = _ROWS // _BLOCK_ROWS
_LOG2E = 1.4426950408889634
_LN2 = 0.6931471805599453


def _ghm_kernel(x_ref, t_ref, out_ref, cnt_ref, sum_ref):
    step = pl.program_id(0)

    @pl.when(step == 0)
    def _init():
        cnt_ref[...] = jnp.zeros_like(cnt_ref)
        sum_ref[...] = jnp.zeros_like(sum_ref)

    zeroh = jnp.zeros((16, 128), jnp.bfloat16)
    oneh = jnp.bfloat16(1.0)
    acc_c = [zeroh] * _BINS
    acc_s = [zeroh] * _BINS

    for i in range(_BLOCK_ROWS // 16):
        for j in range(_COLS // 128):
            x = x_ref[pl.ds(16 * i, 16), pl.ds(128 * j, 128)]
            t = t_ref[pl.ds(16 * i, 16), pl.ds(128 * j, 128)]
            ax = jnp.abs(x)
            e = jnp.exp2(ax * (-_LOG2E))
            one_pe = e + 1.0
            d = 1.0 / one_pe
            sig = jnp.where(x >= 0.0, d, 1.0 - d)
            g = jnp.abs(sig - t)
            binf = jnp.floor(g * _SCALE)
            bce = jnp.maximum(x, 0.0) - x * t + jnp.log2(one_pe) * _LN2
            binh = binf.astype(jnp.bfloat16)
            bceh = bce.astype(jnp.bfloat16)
            for b in range(_BINS):
                mask = binh == jnp.bfloat16(b)
                acc_c[b] = acc_c[b] + jnp.where(mask, oneh, zeroh)
                acc_s[b] = acc_s[b] + jnp.where(mask, bceh, zeroh)

    for b in range(_BINS):
        cnt_ref[pl.ds(16 * b, 16), :] += acc_c[b].astype(jnp.float32)
        sum_ref[pl.ds(16 * b, 16), :] += acc_s[b].astype(jnp.float32)

    @pl.when(step == _N_STEPS - 1)
    def _finish():
        counts = [jnp.sum(cnt_ref[pl.ds(16 * b, 16), :]) for b in range(_BINS)]
        bsums = [jnp.sum(sum_ref[pl.ds(16 * b, 16), :]) for b in range(_BINS)]
        nonempty = jnp.float32(0.0)
        for b in range(_BINS):
            nonempty += jnp.where(counts[b] > 0.0, jnp.float32(1.0),
                                  jnp.float32(0.0))
        loss = jnp.float32(0.0)
        for b in range(_BINS):
            gd = jnp.maximum(counts[b] * nonempty, jnp.float32(0.0001))
            loss += bsums[b] / gd
        out_ref[...] = jnp.full((8, 128), loss, dtype=jnp.float32)


@jax.jit
def kernel(x, target):
    out = pl.pallas_call(
        _ghm_kernel,
        grid=(_N_STEPS,),
        in_specs=[
            pl.BlockSpec((_BLOCK_ROWS, _COLS), lambda i: (i, 0)),
            pl.BlockSpec((_BLOCK_ROWS, _COLS), lambda i: (i, 0)),
        ],
        out_specs=pl.BlockSpec((8, 128), lambda i: (0, 0)),
        out_shape=jax.ShapeDtypeStruct((8, 128), jnp.float32),
        scratch_shapes=[
            pltpu.VMEM((16 * _BINS, 128), jnp.float32),
            pltpu.VMEM((16 * _BINS, 128), jnp.float32),
        ],
    )(x, target)
    return out[0, 0]
